# baseline (speedup 1.0000x reference)
import jax
import jax.numpy as jnp
from jax.experimental import pallas as pl
from jax.experimental.pallas import tpu as pltpu


def _body(x_ref, w_ref, b_ref, o_ref, acc_ref, *, nb, br, d):
    i = pl.program_id(0)

    @pl.when(i == 0)
    def _init():
        acc_ref[...] = jnp.full((8, d), -jnp.inf, dtype=jnp.float32)

    @pl.when(i < nb)
    def _max_phase():
        x = x_ref[...].reshape(br // 8, 8, d)
        acc_ref[...] = jnp.maximum(acc_ref[...], jnp.max(x, axis=0))

    @pl.when(i >= nb)
    def _matmul_phase():
        gmax = jnp.max(acc_ref[...], axis=0, keepdims=True)
        c = jnp.dot(gmax, w_ref[d:, :], preferred_element_type=jnp.float32)
        c = c + b_ref[...]
        y = jnp.dot(x_ref[...], w_ref[:d, :], preferred_element_type=jnp.float32)
        o_ref[...] = jnp.maximum(y + c, 0.0)


def kernel(edge_pred, edge_corner, all_corners, edge_x, image_x, W, b):
    e, d = edge_x.shape
    nb = 40
    assert e % nb == 0 and (e // nb) % 8 == 0
    br = e // nb

    body = lambda *refs: _body(*refs, nb=nb, br=br, d=d)
    out = pl.pallas_call(
        body,
        grid=(2 * nb,),
        in_specs=[
            pl.BlockSpec((br, d), lambda i: (i % nb, 0)),
            pl.BlockSpec((2 * d, d), lambda i: (0, 0)),
            pl.BlockSpec((1, d), lambda i: (0, 0)),
        ],
        out_specs=pl.BlockSpec((br, d), lambda i: (jnp.maximum(i - nb, 0), 0)),
        out_shape=jax.ShapeDtypeStruct((e, d), jnp.float32),
        scratch_shapes=[pltpu.VMEM((8, d), jnp.float32)],
        compiler_params=pltpu.CompilerParams(
            dimension_semantics=("arbitrary",),
        ),
    )(edge_x, W, b.reshape(1, d))
    return out


# speedup vs baseline: 1.0794x; 1.0794x over previous
import jax
import jax.numpy as jnp
from jax.experimental import pallas as pl
from jax.experimental.pallas import tpu as pltpu


def _body(x_ref, w_ref, b_ref, o_ref, acc_ref, *, nb, br, d):
    i = pl.program_id(0)

    @pl.when(i == 0)
    def _init():
        acc_ref[...] = jnp.full((8, d), -jnp.inf, dtype=jnp.float32)

    @pl.when(i < nb)
    def _max_phase():
        x = x_ref[...].reshape(br // 8, 8, d)
        acc_ref[...] = jnp.maximum(acc_ref[...], jnp.max(x, axis=0))

    @pl.when(i >= nb)
    def _matmul_phase():
        gmax = jnp.max(acc_ref[...], axis=0, keepdims=True)
        c = jnp.dot(gmax, w_ref[d:, :], preferred_element_type=jnp.float32)
        c = c + b_ref[...]
        y = jnp.dot(x_ref[...], w_ref[:d, :], preferred_element_type=jnp.float32)
        o_ref[...] = jnp.maximum(y + c, 0.0)


def kernel(edge_pred, edge_corner, all_corners, edge_x, image_x, W, b):
    e, d = edge_x.shape
    nb = 20
    assert e % nb == 0 and (e // nb) % 8 == 0
    br = e // nb

    body = lambda *refs: _body(*refs, nb=nb, br=br, d=d)
    out = pl.pallas_call(
        body,
        grid=(2 * nb,),
        in_specs=[
            pl.BlockSpec((br, d), lambda i: (i % nb, 0)),
            pl.BlockSpec((2 * d, d), lambda i: (0, 0)),
            pl.BlockSpec((1, d), lambda i: (0, 0)),
        ],
        out_specs=pl.BlockSpec((br, d), lambda i: (jnp.maximum(i - nb, 0), 0)),
        out_shape=jax.ShapeDtypeStruct((e, d), jnp.float32),
        scratch_shapes=[pltpu.VMEM((8, d), jnp.float32)],
        compiler_params=pltpu.CompilerParams(
            dimension_semantics=("arbitrary",),
        ),
    )(edge_x, W, b.reshape(1, d))
    return out


# speedup vs baseline: 1.0840x; 1.0043x over previous
import jax
import jax.numpy as jnp
from jax.experimental import pallas as pl
from jax.experimental.pallas import tpu as pltpu


def _body(x_ref, w_ref, b_ref, o_ref, acc_ref, *, nb, br, d):
    i = pl.program_id(0)

    @pl.when(i == 0)
    def _init():
        acc_ref[...] = jnp.full((8, d), -jnp.inf, dtype=jnp.float32)

    @pl.when(i < nb)
    def _max_phase():
        x = x_ref[...].reshape(br // 8, 8, d)
        acc_ref[...] = jnp.maximum(acc_ref[...], jnp.max(x, axis=0))

    @pl.when(i >= nb)
    def _matmul_phase():
        gmax = jnp.max(acc_ref[...], axis=0, keepdims=True)
        c = jnp.dot(gmax, w_ref[d:, :], preferred_element_type=jnp.float32)
        c = c + b_ref[...]
        y = jnp.dot(x_ref[...], w_ref[:d, :], preferred_element_type=jnp.float32)
        o_ref[...] = jnp.maximum(y + c, 0.0)


def kernel(edge_pred, edge_corner, all_corners, edge_x, image_x, W, b):
    e, d = edge_x.shape
    nb = 16
    assert e % nb == 0 and (e // nb) % 8 == 0
    br = e // nb

    body = lambda *refs: _body(*refs, nb=nb, br=br, d=d)
    out = pl.pallas_call(
        body,
        grid=(2 * nb,),
        in_specs=[
            pl.BlockSpec((br, d), lambda i: (i % nb, 0)),
            pl.BlockSpec((2 * d, d), lambda i: (0, 0)),
            pl.BlockSpec((1, d), lambda i: (0, 0)),
        ],
        out_specs=pl.BlockSpec((br, d), lambda i: (jnp.maximum(i - nb, 0), 0)),
        out_shape=jax.ShapeDtypeStruct((e, d), jnp.float32),
        scratch_shapes=[pltpu.VMEM((8, d), jnp.float32)],
        compiler_params=pltpu.CompilerParams(
            dimension_semantics=("arbitrary",),
        ),
    )(edge_x, W, b.reshape(1, d))
    return out
